# baseline (speedup 1.0000x reference)
import functools

import jax
import jax.numpy as jnp
from jax import lax
from jax.experimental import pallas as pl
from jax.experimental.pallas import tpu as pltpu
from jax.experimental.pallas import tpu_sc as plsc

B, T, C = 2, 2048, 768
N = B * T
E = 8
I = 2048
TT = 256
NT = N // TT
MSUB = 256
IB_SH = 512
NI_SH = I // IB_SH
G_MAX = NT + E - 1
NPAD = G_MAX * TT
GM = 32

NC, NS = 2, 16
NW = NC * NS
CHUNK = N // NW


def _dot_nt(a, b):
    return lax.dot_general(a, b, (((1,), (1,)), ((), ())),
                           preferred_element_type=jnp.float32)


def _dot_exact(a, b, dims):
    return lax.dot_general(a, b, (dims, ((), ())),
                           preferred_element_type=jnp.float32,
                           precision=lax.Precision.HIGHEST)


def _router_kernel(x_ref, rw_ref, rb_ref, meta_ref, eid_ref, rank_ref,
                   cnt_out_ref, cnt_ref):
    t = pl.program_id(0)
    logits = _dot_nt(rw_ref[...], x_ref[...])
    logits = jnp.clip(logits + rb_ref[...], -50.0, 50.0)
    probs = jax.nn.sigmoid(logits)
    pmax = jnp.max(probs, axis=0, keepdims=True)
    iota = lax.broadcasted_iota(jnp.int32, (E, TT), 0)
    eid = jnp.min(jnp.where(probs == pmax, iota, E), axis=0,
                  keepdims=True)
    onehot = (iota == eid).astype(jnp.float32)
    prob = jnp.clip(pmax, 1e-8, 1.0 - 1e-8)
    tw = jnp.clip(0.5 + prob + 1e-8, 0.5, 2.0)
    meta_ref[0, 0, :] = (0.5 / tw)[0]
    meta_ref[1, 0, :] = (prob / tw)[0]
    eid_ref[0, 0, :] = eid[0]

    @pl.when(t == 0)
    def _init():
        cnt_ref[...] = jnp.zeros_like(cnt_ref)

    tri = (lax.broadcasted_iota(jnp.int32, (TT, TT), 0)
           < lax.broadcasted_iota(jnp.int32, (TT, TT), 1)).astype(jnp.float32)
    rank_in = _dot_exact(onehot, tri, ((1,), (0,)))
    carry = cnt_ref[...]
    carry_sel = _dot_exact(carry, onehot, ((0,), (0,)))
    rank_tok = jnp.sum(onehot * rank_in, axis=0, keepdims=True) + carry_sel
    rank_ref[0, 0, :] = rank_tok[0]
    cnt_ref[...] = carry + jnp.sum(onehot, axis=1, keepdims=True)
    cnt_out_ref[...] = cnt_ref[...]


def _positions_kernel(eid_ref, rank_ref, cnt_ref, pos_ref, blk_ref):
    t = pl.program_id(0)
    c = cnt_ref[...]
    pc = jnp.ceil(c * (1.0 / TT)) * TT
    tril = (lax.broadcasted_iota(jnp.int32, (E, E), 1)
            < lax.broadcasted_iota(jnp.int32, (E, E), 0)).astype(jnp.float32)
    po = _dot_exact(tril, pc, ((1,), (0,)))
    eid = eid_ref[0]
    iota = lax.broadcasted_iota(jnp.int32, (E, TT), 0)
    onehot = (iota == eid).astype(jnp.float32)
    po_sel = _dot_exact(po, onehot, ((0,), (0,)))
    pos = po_sel + rank_ref[0]
    pos_ref[0, 0, :] = pos[0].astype(jnp.int32)

    @pl.when(t == 0)
    def _blockmap():
        gstart = (lax.broadcasted_iota(jnp.int32, (1, GM), 1)
                  .astype(jnp.float32) * TT)
        cmp = (po <= gstart).astype(jnp.float32)
        n_le = jnp.sum(cmp, axis=0, keepdims=True)
        total = jnp.sum(pc)
        active = gstart < total
        eg = jnp.where(active, n_le.astype(jnp.int32) - 1, 0)
        blk_ref[pl.ds(0, 1), :] = eg
        blk_ref[pl.ds(1, 1), :] = active.astype(jnp.int32)


def _scatter_body(x_hbm, pos_hbm, out_hbm, idx_v, rows_v, sem):
    wid = lax.axis_index("s") * NC + lax.axis_index("c")
    base = wid * CHUNK
    pltpu.sync_copy(pos_hbm.at[pl.ds(base, CHUNK)], idx_v)
    pltpu.sync_copy(x_hbm.at[pl.ds(base, CHUNK)], rows_v)
    pltpu.async_copy(rows_v, out_hbm.at[idx_v], sem).wait()


def _gather_body(ys_hbm, pos_hbm, out_hbm, idx_v, rows_v, sem):
    wid = lax.axis_index("s") * NC + lax.axis_index("c")
    base = wid * CHUNK
    pltpu.sync_copy(pos_hbm.at[pl.ds(base, CHUNK)], idx_v)
    pltpu.async_copy(ys_hbm.at[idx_v], rows_v, sem).wait()
    pltpu.sync_copy(rows_v, out_hbm.at[pl.ds(base, CHUNK)])


def _sc_mesh():
    return plsc.VectorSubcoreMesh(core_axis_name="c", subcore_axis_name="s",
                                  num_cores=NC, num_subcores=NS)


def _sc_scratch():
    return [
        pltpu.VMEM((CHUNK,), jnp.int32),
        pltpu.VMEM((CHUNK, C), jnp.float32),
        pltpu.SemaphoreType.DMA,
    ]


def _sc_scatter(x_flat, pos):
    f = pl.kernel(_scatter_body,
                  out_type=jax.ShapeDtypeStruct((NPAD, C), jnp.float32),
                  mesh=_sc_mesh(), scratch_types=_sc_scratch())
    return f(x_flat, pos)


def _sc_gather(ys, pos):
    f = pl.kernel(_gather_body,
                  out_type=jax.ShapeDtypeStruct((N, C), jnp.float32),
                  mesh=_sc_mesh(), scratch_types=_sc_scratch())
    return f(ys, pos)


def _swiglu_block(xb, w1, w3, w2):
    h1 = _dot_nt(xb, w1.astype(jnp.bfloat16))
    h3 = _dot_nt(xb, w3.astype(jnp.bfloat16))
    g = (jax.nn.silu(h1) * h3).astype(jnp.bfloat16)
    return _dot_nt(g, w2.astype(jnp.bfloat16))


def _grouped_kernel(be_ref, ba_ref, xs_ref, w1_ref, w3_ref, w2_ref, ys_ref):
    g = pl.program_id(0)

    @pl.when(ba_ref[g] == 0)
    def _inactive():
        ys_ref[...] = jnp.zeros_like(ys_ref)

    @pl.when(ba_ref[g] == 1)
    def _compute():
        for ms in range(TT // MSUB):
            sl = pl.ds(ms * MSUB, MSUB)
            xb = xs_ref[sl, :].astype(jnp.bfloat16)
            ys_ref[sl, :] = _swiglu_block(xb, w1_ref[0], w3_ref[0],
                                          w2_ref[0])


def _shared_kernel(x_ref, w1_ref, w3_ref, w2_ref, o_ref):
    i = pl.program_id(0)

    @pl.when(i == 0)
    def _init():
        o_ref[...] = jnp.zeros_like(o_ref)

    xb = x_ref[...].astype(jnp.bfloat16)
    o_ref[...] += _swiglu_block(xb, w1_ref[...], w3_ref[...], w2_ref[...])


def _combine_kernel(meta_ref, sh_ref, ysu_ref, o_ref):
    ssh = meta_ref[0, 0, :]
    ssw = meta_ref[1, 0, :]
    o_ref[...] = ssh[:, None] * sh_ref[...] + ssw[:, None] * ysu_ref[...]


def kernel(x, router_w, routing_bias, sw1, sw2, sw3, ew1, ew2, ew3):
    x_flat = x.reshape(N, C)
    rb = routing_bias.reshape(E, 1)

    meta, eid, rank, counts = pl.pallas_call(
        _router_kernel,
        grid=(NT,),
        in_specs=[
            pl.BlockSpec((TT, C), lambda t: (t, 0)),
            pl.BlockSpec((E, C), lambda t: (0, 0)),
            pl.BlockSpec((E, 1), lambda t: (0, 0)),
        ],
        out_specs=[
            pl.BlockSpec((2, 1, TT), lambda t: (0, 0, t)),
            pl.BlockSpec((1, 1, TT), lambda t: (0, 0, t)),
            pl.BlockSpec((1, 1, TT), lambda t: (0, 0, t)),
            pl.BlockSpec((E, 1), lambda t: (0, 0)),
        ],
        out_shape=[
            jax.ShapeDtypeStruct((2, 1, N), jnp.float32),
            jax.ShapeDtypeStruct((1, 1, N), jnp.int32),
            jax.ShapeDtypeStruct((1, 1, N), jnp.float32),
            jax.ShapeDtypeStruct((E, 1), jnp.float32),
        ],
        scratch_shapes=[pltpu.VMEM((E, 1), jnp.float32)],
    )(x_flat, router_w, rb)

    pos3, blk = pl.pallas_call(
        _positions_kernel,
        grid=(NT,),
        in_specs=[
            pl.BlockSpec((1, 1, TT), lambda t: (0, 0, t)),
            pl.BlockSpec((1, 1, TT), lambda t: (0, 0, t)),
            pl.BlockSpec((E, 1), lambda t: (0, 0)),
        ],
        out_specs=[
            pl.BlockSpec((1, 1, TT), lambda t: (0, 0, t)),
            pl.BlockSpec((2, GM), lambda t: (0, 0)),
        ],
        out_shape=[
            jax.ShapeDtypeStruct((1, 1, N), jnp.int32),
            jax.ShapeDtypeStruct((2, GM), jnp.int32),
        ],
    )(eid, rank, counts)

    pos = pos3.reshape(N)
    be = blk[0]
    ba = blk[1]

    xs = _sc_scatter(x_flat, pos)

    ys = pl.pallas_call(
        _grouped_kernel,
        grid_spec=pltpu.PrefetchScalarGridSpec(
            num_scalar_prefetch=2,
            grid=(G_MAX,),
            in_specs=[
                pl.BlockSpec((TT, C), lambda g, be, ba: (g, 0)),
                pl.BlockSpec((1, I, C), lambda g, be, ba: (be[g], 0, 0)),
                pl.BlockSpec((1, I, C), lambda g, be, ba: (be[g], 0, 0)),
                pl.BlockSpec((1, C, I), lambda g, be, ba: (be[g], 0, 0)),
            ],
            out_specs=pl.BlockSpec((TT, C), lambda g, be, ba: (g, 0)),
        ),
        out_shape=jax.ShapeDtypeStruct((NPAD, C), jnp.float32),
    )(be, ba, xs, ew1, ew3, ew2)

    shared = pl.pallas_call(
        _shared_kernel,
        grid=(NI_SH,),
        in_specs=[
            pl.BlockSpec((N, C), lambda i: (0, 0)),
            pl.BlockSpec((IB_SH, C), lambda i: (i, 0)),
            pl.BlockSpec((IB_SH, C), lambda i: (i, 0)),
            pl.BlockSpec((C, IB_SH), lambda i: (0, i)),
        ],
        out_specs=pl.BlockSpec((N, C), lambda i: (0, 0)),
        out_shape=jax.ShapeDtypeStruct((N, C), jnp.float32),
    )(x_flat, sw1, sw3, sw2)

    ysu = _sc_gather(ys, pos)

    out = pl.pallas_call(
        _combine_kernel,
        grid=(NT,),
        in_specs=[
            pl.BlockSpec((2, 1, TT), lambda t: (0, 0, t)),
            pl.BlockSpec((TT, C), lambda t: (t, 0)),
            pl.BlockSpec((TT, C), lambda t: (t, 0)),
        ],
        out_specs=pl.BlockSpec((TT, C), lambda t: (t, 0)),
        out_shape=jax.ShapeDtypeStruct((N, C), jnp.float32),
    )(meta, shared, ysu)

    return out.reshape(B, T, C)


# speedup vs baseline: 1.1355x; 1.1355x over previous
import functools

import jax
import jax.numpy as jnp
from jax import lax
from jax.experimental import pallas as pl
from jax.experimental.pallas import tpu as pltpu
from jax.experimental.pallas import tpu_sc as plsc

B, T, C = 2, 2048, 768
N = B * T
E = 8
I = 2048
TT = 512
NT = N // TT
MSUB = 256
IB_SH = 512
NI_SH = I // IB_SH
G_MAX = NT + E - 1
NPAD = G_MAX * TT
GM = 32

NC, NS = 2, 16
NW = NC * NS
CHUNK = N // NW


def _dot_nt(a, b):
    return lax.dot_general(a, b, (((1,), (1,)), ((), ())),
                           preferred_element_type=jnp.float32)


def _dot_exact(a, b, dims):
    return lax.dot_general(a, b, (dims, ((), ())),
                           preferred_element_type=jnp.float32,
                           precision=lax.Precision.HIGHEST)


def _router_kernel(x_ref, rw_ref, rb_ref, meta_ref, eid_ref, rank_ref,
                   cnt_out_ref, cnt_ref):
    t = pl.program_id(0)
    logits = _dot_nt(rw_ref[...], x_ref[...])
    logits = jnp.clip(logits + rb_ref[...], -50.0, 50.0)
    probs = jax.nn.sigmoid(logits)
    pmax = jnp.max(probs, axis=0, keepdims=True)
    iota = lax.broadcasted_iota(jnp.int32, (E, TT), 0)
    eid = jnp.min(jnp.where(probs == pmax, iota, E), axis=0,
                  keepdims=True)
    onehot = (iota == eid).astype(jnp.float32)
    prob = jnp.clip(pmax, 1e-8, 1.0 - 1e-8)
    tw = jnp.clip(0.5 + prob + 1e-8, 0.5, 2.0)
    meta_ref[0, 0, :] = (0.5 / tw)[0]
    meta_ref[1, 0, :] = (prob / tw)[0]
    eid_ref[0, 0, :] = eid[0]

    @pl.when(t == 0)
    def _init():
        cnt_ref[...] = jnp.zeros_like(cnt_ref)

    tri = (lax.broadcasted_iota(jnp.int32, (TT, TT), 0)
           < lax.broadcasted_iota(jnp.int32, (TT, TT), 1)).astype(jnp.float32)
    rank_in = _dot_exact(onehot, tri, ((1,), (0,)))
    carry = cnt_ref[...]
    carry_sel = _dot_exact(carry, onehot, ((0,), (0,)))
    rank_tok = jnp.sum(onehot * rank_in, axis=0, keepdims=True) + carry_sel
    rank_ref[0, 0, :] = rank_tok[0]
    cnt_ref[...] = carry + jnp.sum(onehot, axis=1, keepdims=True)
    cnt_out_ref[...] = cnt_ref[...]


def _positions_kernel(eid_ref, rank_ref, cnt_ref, pos_ref, blk_ref):
    t = pl.program_id(0)
    c = cnt_ref[...]
    pc = jnp.ceil(c * (1.0 / TT)) * TT
    tril = (lax.broadcasted_iota(jnp.int32, (E, E), 1)
            < lax.broadcasted_iota(jnp.int32, (E, E), 0)).astype(jnp.float32)
    po = _dot_exact(tril, pc, ((1,), (0,)))
    eid = eid_ref[0]
    iota = lax.broadcasted_iota(jnp.int32, (E, TT), 0)
    onehot = (iota == eid).astype(jnp.float32)
    po_sel = _dot_exact(po, onehot, ((0,), (0,)))
    pos = po_sel + rank_ref[0]
    pos_ref[0, 0, :] = pos[0].astype(jnp.int32)

    @pl.when(t == 0)
    def _blockmap():
        gstart = (lax.broadcasted_iota(jnp.int32, (1, GM), 1)
                  .astype(jnp.float32) * TT)
        cmp = (po <= gstart).astype(jnp.float32)
        n_le = jnp.sum(cmp, axis=0, keepdims=True)
        total = jnp.sum(pc)
        active = gstart < total
        eg = jnp.where(active, n_le.astype(jnp.int32) - 1, 0)
        blk_ref[pl.ds(0, 1), :] = eg
        blk_ref[pl.ds(1, 1), :] = active.astype(jnp.int32)


def _scatter_body(x_hbm, pos_hbm, out_hbm, idx_v, rows_v, sem):
    wid = lax.axis_index("s") * NC + lax.axis_index("c")
    base = wid * CHUNK
    pltpu.sync_copy(pos_hbm.at[pl.ds(base, CHUNK)], idx_v)
    pltpu.sync_copy(x_hbm.at[pl.ds(base, CHUNK)], rows_v)
    pltpu.async_copy(rows_v, out_hbm.at[idx_v], sem).wait()


def _gather_body(ys_hbm, pos_hbm, out_hbm, idx_v, rows_v, sem):
    wid = lax.axis_index("s") * NC + lax.axis_index("c")
    base = wid * CHUNK
    pltpu.sync_copy(pos_hbm.at[pl.ds(base, CHUNK)], idx_v)
    pltpu.async_copy(ys_hbm.at[idx_v], rows_v, sem).wait()
    pltpu.sync_copy(rows_v, out_hbm.at[pl.ds(base, CHUNK)])


def _sc_mesh():
    return plsc.VectorSubcoreMesh(core_axis_name="c", subcore_axis_name="s",
                                  num_cores=NC, num_subcores=NS)


def _sc_scratch():
    return [
        pltpu.VMEM((CHUNK,), jnp.int32),
        pltpu.VMEM((CHUNK, C), jnp.float32),
        pltpu.SemaphoreType.DMA,
    ]


def _sc_scatter(x_flat, pos):
    f = pl.kernel(_scatter_body,
                  out_type=jax.ShapeDtypeStruct((NPAD, C), jnp.float32),
                  mesh=_sc_mesh(), scratch_types=_sc_scratch())
    return f(x_flat, pos)


def _sc_gather(ys, pos):
    f = pl.kernel(_gather_body,
                  out_type=jax.ShapeDtypeStruct((N, C), jnp.float32),
                  mesh=_sc_mesh(), scratch_types=_sc_scratch())
    return f(ys, pos)


def _swiglu_block(xb, w1, w3, w2):
    h1 = _dot_nt(xb, w1.astype(jnp.bfloat16))
    h3 = _dot_nt(xb, w3.astype(jnp.bfloat16))
    g = (jax.nn.silu(h1) * h3).astype(jnp.bfloat16)
    return _dot_nt(g, w2.astype(jnp.bfloat16))


def _grouped_kernel(be_ref, ba_ref, xs_ref, w1_ref, w3_ref, w2_ref, ys_ref):
    g = pl.program_id(0)

    @pl.when(ba_ref[g] == 0)
    def _inactive():
        ys_ref[...] = jnp.zeros_like(ys_ref)

    @pl.when(ba_ref[g] == 1)
    def _compute():
        for ms in range(TT // MSUB):
            sl = pl.ds(ms * MSUB, MSUB)
            xb = xs_ref[sl, :].astype(jnp.bfloat16)
            ys_ref[sl, :] = _swiglu_block(xb, w1_ref[0], w3_ref[0],
                                          w2_ref[0])


def _shared_kernel(x_ref, w1_ref, w3_ref, w2_ref, o_ref):
    i = pl.program_id(0)

    @pl.when(i == 0)
    def _init():
        o_ref[...] = jnp.zeros_like(o_ref)

    xb = x_ref[...].astype(jnp.bfloat16)
    o_ref[...] += _swiglu_block(xb, w1_ref[...], w3_ref[...], w2_ref[...])


def _combine_kernel(meta_ref, sh_ref, ysu_ref, o_ref):
    ssh = meta_ref[0, 0, :]
    ssw = meta_ref[1, 0, :]
    o_ref[...] = ssh[:, None] * sh_ref[...] + ssw[:, None] * ysu_ref[...]


def kernel(x, router_w, routing_bias, sw1, sw2, sw3, ew1, ew2, ew3):
    x_flat = x.reshape(N, C)
    rb = routing_bias.reshape(E, 1)

    meta, eid, rank, counts = pl.pallas_call(
        _router_kernel,
        grid=(NT,),
        in_specs=[
            pl.BlockSpec((TT, C), lambda t: (t, 0)),
            pl.BlockSpec((E, C), lambda t: (0, 0)),
            pl.BlockSpec((E, 1), lambda t: (0, 0)),
        ],
        out_specs=[
            pl.BlockSpec((2, 1, TT), lambda t: (0, 0, t)),
            pl.BlockSpec((1, 1, TT), lambda t: (0, 0, t)),
            pl.BlockSpec((1, 1, TT), lambda t: (0, 0, t)),
            pl.BlockSpec((E, 1), lambda t: (0, 0)),
        ],
        out_shape=[
            jax.ShapeDtypeStruct((2, 1, N), jnp.float32),
            jax.ShapeDtypeStruct((1, 1, N), jnp.int32),
            jax.ShapeDtypeStruct((1, 1, N), jnp.float32),
            jax.ShapeDtypeStruct((E, 1), jnp.float32),
        ],
        scratch_shapes=[pltpu.VMEM((E, 1), jnp.float32)],
    )(x_flat, router_w, rb)

    pos3, blk = pl.pallas_call(
        _positions_kernel,
        grid=(NT,),
        in_specs=[
            pl.BlockSpec((1, 1, TT), lambda t: (0, 0, t)),
            pl.BlockSpec((1, 1, TT), lambda t: (0, 0, t)),
            pl.BlockSpec((E, 1), lambda t: (0, 0)),
        ],
        out_specs=[
            pl.BlockSpec((1, 1, TT), lambda t: (0, 0, t)),
            pl.BlockSpec((2, GM), lambda t: (0, 0)),
        ],
        out_shape=[
            jax.ShapeDtypeStruct((1, 1, N), jnp.int32),
            jax.ShapeDtypeStruct((2, GM), jnp.int32),
        ],
    )(eid, rank, counts)

    pos = pos3.reshape(N)
    be = blk[0]
    ba = blk[1]

    xs = _sc_scatter(x_flat, pos)

    ys = pl.pallas_call(
        _grouped_kernel,
        grid_spec=pltpu.PrefetchScalarGridSpec(
            num_scalar_prefetch=2,
            grid=(G_MAX,),
            in_specs=[
                pl.BlockSpec((TT, C), lambda g, be, ba: (g, 0)),
                pl.BlockSpec((1, I, C), lambda g, be, ba: (be[g], 0, 0)),
                pl.BlockSpec((1, I, C), lambda g, be, ba: (be[g], 0, 0)),
                pl.BlockSpec((1, C, I), lambda g, be, ba: (be[g], 0, 0)),
            ],
            out_specs=pl.BlockSpec((TT, C), lambda g, be, ba: (g, 0)),
        ),
        out_shape=jax.ShapeDtypeStruct((NPAD, C), jnp.float32),
    )(be, ba, xs, ew1, ew3, ew2)

    shared = pl.pallas_call(
        _shared_kernel,
        grid=(NI_SH,),
        in_specs=[
            pl.BlockSpec((N, C), lambda i: (0, 0)),
            pl.BlockSpec((IB_SH, C), lambda i: (i, 0)),
            pl.BlockSpec((IB_SH, C), lambda i: (i, 0)),
            pl.BlockSpec((C, IB_SH), lambda i: (0, i)),
        ],
        out_specs=pl.BlockSpec((N, C), lambda i: (0, 0)),
        out_shape=jax.ShapeDtypeStruct((N, C), jnp.float32),
    )(x_flat, sw1, sw3, sw2)

    ysu = _sc_gather(ys, pos)

    out = pl.pallas_call(
        _combine_kernel,
        grid=(NT,),
        in_specs=[
            pl.BlockSpec((2, 1, TT), lambda t: (0, 0, t)),
            pl.BlockSpec((TT, C), lambda t: (t, 0)),
            pl.BlockSpec((TT, C), lambda t: (t, 0)),
        ],
        out_specs=pl.BlockSpec((TT, C), lambda t: (t, 0)),
        out_shape=jax.ShapeDtypeStruct((N, C), jnp.float32),
    )(meta, shared, ysu)

    return out.reshape(B, T, C)


# speedup vs baseline: 1.1418x; 1.0055x over previous
import functools

import jax
import jax.numpy as jnp
from jax import lax
from jax.experimental import pallas as pl
from jax.experimental.pallas import tpu as pltpu
from jax.experimental.pallas import tpu_sc as plsc

B, T, C = 2, 2048, 768
N = B * T
E = 8
I = 2048
TT = 512
NT = N // TT
MSUB = 512
IB_SH = 512
NI_SH = I // IB_SH
G_MAX = NT + E - 1
NPAD = G_MAX * TT
GM = 32

NC, NS = 2, 16
NW = NC * NS
CHUNK = N // NW


def _dot_nt(a, b):
    return lax.dot_general(a, b, (((1,), (1,)), ((), ())),
                           preferred_element_type=jnp.float32)


def _dot_exact(a, b, dims):
    return lax.dot_general(a, b, (dims, ((), ())),
                           preferred_element_type=jnp.float32,
                           precision=lax.Precision.HIGHEST)


def _router_kernel(x_ref, rw_ref, rb_ref, meta_ref, eid_ref, rank_ref,
                   cnt_out_ref, cnt_ref):
    t = pl.program_id(0)
    logits = _dot_nt(rw_ref[...], x_ref[...])
    logits = jnp.clip(logits + rb_ref[...], -50.0, 50.0)
    probs = jax.nn.sigmoid(logits)
    pmax = jnp.max(probs, axis=0, keepdims=True)
    iota = lax.broadcasted_iota(jnp.int32, (E, TT), 0)
    eid = jnp.min(jnp.where(probs == pmax, iota, E), axis=0,
                  keepdims=True)
    onehot = (iota == eid).astype(jnp.float32)
    prob = jnp.clip(pmax, 1e-8, 1.0 - 1e-8)
    tw = jnp.clip(0.5 + prob + 1e-8, 0.5, 2.0)
    meta_ref[0, 0, :] = (0.5 / tw)[0]
    meta_ref[1, 0, :] = (prob / tw)[0]
    eid_ref[0, 0, :] = eid[0]

    @pl.when(t == 0)
    def _init():
        cnt_ref[...] = jnp.zeros_like(cnt_ref)

    tri = (lax.broadcasted_iota(jnp.int32, (TT, TT), 0)
           < lax.broadcasted_iota(jnp.int32, (TT, TT), 1)).astype(jnp.float32)
    rank_in = _dot_exact(onehot, tri, ((1,), (0,)))
    carry = cnt_ref[...]
    carry_sel = _dot_exact(carry, onehot, ((0,), (0,)))
    rank_tok = jnp.sum(onehot * rank_in, axis=0, keepdims=True) + carry_sel
    rank_ref[0, 0, :] = rank_tok[0]
    cnt_ref[...] = carry + jnp.sum(onehot, axis=1, keepdims=True)
    cnt_out_ref[...] = cnt_ref[...]


def _positions_kernel(eid_ref, rank_ref, cnt_ref, pos_ref, blk_ref):
    t = pl.program_id(0)
    c = cnt_ref[...]
    pc = jnp.ceil(c * (1.0 / TT)) * TT
    tril = (lax.broadcasted_iota(jnp.int32, (E, E), 1)
            < lax.broadcasted_iota(jnp.int32, (E, E), 0)).astype(jnp.float32)
    po = _dot_exact(tril, pc, ((1,), (0,)))
    eid = eid_ref[0]
    iota = lax.broadcasted_iota(jnp.int32, (E, TT), 0)
    onehot = (iota == eid).astype(jnp.float32)
    po_sel = _dot_exact(po, onehot, ((0,), (0,)))
    pos = po_sel + rank_ref[0]
    pos_ref[0, 0, :] = pos[0].astype(jnp.int32)

    @pl.when(t == 0)
    def _blockmap():
        gstart = (lax.broadcasted_iota(jnp.int32, (1, GM), 1)
                  .astype(jnp.float32) * TT)
        cmp = (po <= gstart).astype(jnp.float32)
        n_le = jnp.sum(cmp, axis=0, keepdims=True)
        total = jnp.sum(pc)
        active = gstart < total
        eg = jnp.where(active, n_le.astype(jnp.int32) - 1, 0)
        blk_ref[pl.ds(0, 1), :] = eg
        blk_ref[pl.ds(1, 1), :] = active.astype(jnp.int32)


def _scatter_body(x_hbm, pos_hbm, out_hbm, idx_v, rows_v, sem):
    wid = lax.axis_index("s") * NC + lax.axis_index("c")
    base = wid * CHUNK
    pltpu.sync_copy(pos_hbm.at[pl.ds(base, CHUNK)], idx_v)
    pltpu.sync_copy(x_hbm.at[pl.ds(base, CHUNK)], rows_v)
    pltpu.async_copy(rows_v, out_hbm.at[idx_v], sem).wait()


def _gather_body(ys_hbm, pos_hbm, out_hbm, idx_v, rows_v, sem):
    wid = lax.axis_index("s") * NC + lax.axis_index("c")
    base = wid * CHUNK
    pltpu.sync_copy(pos_hbm.at[pl.ds(base, CHUNK)], idx_v)
    pltpu.async_copy(ys_hbm.at[idx_v], rows_v, sem).wait()
    pltpu.sync_copy(rows_v, out_hbm.at[pl.ds(base, CHUNK)])


def _sc_mesh():
    return plsc.VectorSubcoreMesh(core_axis_name="c", subcore_axis_name="s",
                                  num_cores=NC, num_subcores=NS)


def _sc_scratch():
    return [
        pltpu.VMEM((CHUNK,), jnp.int32),
        pltpu.VMEM((CHUNK, C), jnp.float32),
        pltpu.SemaphoreType.DMA,
    ]


def _sc_scatter(x_flat, pos):
    f = pl.kernel(_scatter_body,
                  out_type=jax.ShapeDtypeStruct((NPAD, C), jnp.float32),
                  mesh=_sc_mesh(), scratch_types=_sc_scratch())
    return f(x_flat, pos)


def _sc_gather(ys, pos):
    f = pl.kernel(_gather_body,
                  out_type=jax.ShapeDtypeStruct((N, C), jnp.float32),
                  mesh=_sc_mesh(), scratch_types=_sc_scratch())
    return f(ys, pos)


def _swiglu_block(xb, w1, w3, w2):
    h1 = _dot_nt(xb, w1.astype(jnp.bfloat16))
    h3 = _dot_nt(xb, w3.astype(jnp.bfloat16))
    g = (jax.nn.silu(h1) * h3).astype(jnp.bfloat16)
    return _dot_nt(g, w2.astype(jnp.bfloat16))


def _grouped_kernel(be_ref, ba_ref, xs_ref, w1_ref, w3_ref, w2_ref, ys_ref):
    g = pl.program_id(0)

    @pl.when(ba_ref[g] == 0)
    def _inactive():
        ys_ref[...] = jnp.zeros_like(ys_ref)

    @pl.when(ba_ref[g] == 1)
    def _compute():
        for ms in range(TT // MSUB):
            sl = pl.ds(ms * MSUB, MSUB)
            xb = xs_ref[sl, :].astype(jnp.bfloat16)
            ys_ref[sl, :] = _swiglu_block(xb, w1_ref[0], w3_ref[0],
                                          w2_ref[0])


def _shared_kernel(x_ref, w1_ref, w3_ref, w2_ref, o_ref):
    i = pl.program_id(0)

    @pl.when(i == 0)
    def _init():
        o_ref[...] = jnp.zeros_like(o_ref)

    xb = x_ref[...].astype(jnp.bfloat16)
    o_ref[...] += _swiglu_block(xb, w1_ref[...], w3_ref[...], w2_ref[...])


def _combine_kernel(meta_ref, sh_ref, ysu_ref, o_ref):
    ssh = meta_ref[0, 0, :]
    ssw = meta_ref[1, 0, :]
    o_ref[...] = ssh[:, None] * sh_ref[...] + ssw[:, None] * ysu_ref[...]


def kernel(x, router_w, routing_bias, sw1, sw2, sw3, ew1, ew2, ew3):
    x_flat = x.reshape(N, C)
    rb = routing_bias.reshape(E, 1)

    meta, eid, rank, counts = pl.pallas_call(
        _router_kernel,
        grid=(NT,),
        in_specs=[
            pl.BlockSpec((TT, C), lambda t: (t, 0)),
            pl.BlockSpec((E, C), lambda t: (0, 0)),
            pl.BlockSpec((E, 1), lambda t: (0, 0)),
        ],
        out_specs=[
            pl.BlockSpec((2, 1, TT), lambda t: (0, 0, t)),
            pl.BlockSpec((1, 1, TT), lambda t: (0, 0, t)),
            pl.BlockSpec((1, 1, TT), lambda t: (0, 0, t)),
            pl.BlockSpec((E, 1), lambda t: (0, 0)),
        ],
        out_shape=[
            jax.ShapeDtypeStruct((2, 1, N), jnp.float32),
            jax.ShapeDtypeStruct((1, 1, N), jnp.int32),
            jax.ShapeDtypeStruct((1, 1, N), jnp.float32),
            jax.ShapeDtypeStruct((E, 1), jnp.float32),
        ],
        scratch_shapes=[pltpu.VMEM((E, 1), jnp.float32)],
    )(x_flat, router_w, rb)

    pos3, blk = pl.pallas_call(
        _positions_kernel,
        grid=(NT,),
        in_specs=[
            pl.BlockSpec((1, 1, TT), lambda t: (0, 0, t)),
            pl.BlockSpec((1, 1, TT), lambda t: (0, 0, t)),
            pl.BlockSpec((E, 1), lambda t: (0, 0)),
        ],
        out_specs=[
            pl.BlockSpec((1, 1, TT), lambda t: (0, 0, t)),
            pl.BlockSpec((2, GM), lambda t: (0, 0)),
        ],
        out_shape=[
            jax.ShapeDtypeStruct((1, 1, N), jnp.int32),
            jax.ShapeDtypeStruct((2, GM), jnp.int32),
        ],
    )(eid, rank, counts)

    pos = pos3.reshape(N)
    be = blk[0]
    ba = blk[1]

    xs = _sc_scatter(x_flat, pos)

    ys = pl.pallas_call(
        _grouped_kernel,
        grid_spec=pltpu.PrefetchScalarGridSpec(
            num_scalar_prefetch=2,
            grid=(G_MAX,),
            in_specs=[
                pl.BlockSpec((TT, C), lambda g, be, ba: (g, 0)),
                pl.BlockSpec((1, I, C), lambda g, be, ba: (be[g], 0, 0)),
                pl.BlockSpec((1, I, C), lambda g, be, ba: (be[g], 0, 0)),
                pl.BlockSpec((1, C, I), lambda g, be, ba: (be[g], 0, 0)),
            ],
            out_specs=pl.BlockSpec((TT, C), lambda g, be, ba: (g, 0)),
        ),
        out_shape=jax.ShapeDtypeStruct((NPAD, C), jnp.float32),
    )(be, ba, xs, ew1, ew3, ew2)

    shared = pl.pallas_call(
        _shared_kernel,
        grid=(NI_SH,),
        in_specs=[
            pl.BlockSpec((N, C), lambda i: (0, 0)),
            pl.BlockSpec((IB_SH, C), lambda i: (i, 0)),
            pl.BlockSpec((IB_SH, C), lambda i: (i, 0)),
            pl.BlockSpec((C, IB_SH), lambda i: (0, i)),
        ],
        out_specs=pl.BlockSpec((N, C), lambda i: (0, 0)),
        out_shape=jax.ShapeDtypeStruct((N, C), jnp.float32),
    )(x_flat, sw1, sw3, sw2)

    ysu = _sc_gather(ys, pos)

    out = pl.pallas_call(
        _combine_kernel,
        grid=(NT,),
        in_specs=[
            pl.BlockSpec((2, 1, TT), lambda t: (0, 0, t)),
            pl.BlockSpec((TT, C), lambda t: (t, 0)),
            pl.BlockSpec((TT, C), lambda t: (t, 0)),
        ],
        out_specs=pl.BlockSpec((TT, C), lambda t: (t, 0)),
        out_shape=jax.ShapeDtypeStruct((N, C), jnp.float32),
    )(meta, shared, ysu)

    return out.reshape(B, T, C)


# speedup vs baseline: 1.1510x; 1.0081x over previous
import functools

import jax
import jax.numpy as jnp
from jax import lax
from jax.experimental import pallas as pl
from jax.experimental.pallas import tpu as pltpu
from jax.experimental.pallas import tpu_sc as plsc

B, T, C = 2, 2048, 768
N = B * T
E = 8
I = 2048
TT = 512
NT = N // TT
MSUB = 512
IB_SH = 512
NI_SH = I // IB_SH
G_MAX = NT + E - 1
NPAD = G_MAX * TT
GM = 32

NC, NS = 2, 16
NW = NC * NS
CHUNK = N // NW


def _dot_nt(a, b):
    return lax.dot_general(a, b, (((1,), (1,)), ((), ())),
                           preferred_element_type=jnp.float32)


def _dot_exact(a, b, dims):
    return lax.dot_general(a, b, (dims, ((), ())),
                           preferred_element_type=jnp.float32,
                           precision=lax.Precision.HIGHEST)


def _router_kernel(x_ref, rw_ref, rb_ref, meta_ref, pos_ref, blk_ref,
                   cnt_ref, eid_s, rank_s):
    p = pl.program_id(0)
    t = pl.program_id(1)
    iota = lax.broadcasted_iota(jnp.int32, (E, TT), 0)

    @pl.when(p == 0)
    def _route():
        logits = _dot_nt(rw_ref[...], x_ref[...])
        logits = jnp.clip(logits + rb_ref[...], -50.0, 50.0)
        probs = jax.nn.sigmoid(logits)
        pmax = jnp.max(probs, axis=0, keepdims=True)
        eid = jnp.min(jnp.where(probs == pmax, iota, E), axis=0,
                      keepdims=True)
        onehot = (iota == eid).astype(jnp.float32)
        prob = jnp.clip(pmax, 1e-8, 1.0 - 1e-8)
        tw = jnp.clip(0.5 + prob + 1e-8, 0.5, 2.0)
        meta_ref[0, 0, :] = (0.5 / tw)[0]
        meta_ref[1, 0, :] = (prob / tw)[0]
        eid_s[0, pl.ds(t * TT, TT)] = eid[0]

        @pl.when(t == 0)
        def _init():
            cnt_ref[...] = jnp.zeros_like(cnt_ref)

        tri = (lax.broadcasted_iota(jnp.int32, (TT, TT), 0)
               < lax.broadcasted_iota(jnp.int32, (TT, TT), 1)
               ).astype(jnp.float32)
        rank_in = _dot_exact(onehot, tri, ((1,), (0,)))
        carry = cnt_ref[...]
        carry_sel = _dot_exact(carry, onehot, ((0,), (0,)))
        rank_tok = jnp.sum(onehot * rank_in, axis=0, keepdims=True) \
            + carry_sel
        rank_s[0, pl.ds(t * TT, TT)] = rank_tok[0]
        cnt_ref[...] = carry + jnp.sum(onehot, axis=1, keepdims=True)

    @pl.when(p == 1)
    def _positions():
        c = cnt_ref[...]
        pc = jnp.ceil(c * (1.0 / TT)) * TT
        tril = (lax.broadcasted_iota(jnp.int32, (E, E), 1)
                < lax.broadcasted_iota(jnp.int32, (E, E), 0)
                ).astype(jnp.float32)
        po = _dot_exact(tril, pc, ((1,), (0,)))
        eid = eid_s[0:1, pl.ds(t * TT, TT)]
        onehot = (iota == eid).astype(jnp.float32)
        po_sel = _dot_exact(po, onehot, ((0,), (0,)))
        pos = po_sel + rank_s[0:1, pl.ds(t * TT, TT)]
        pos_ref[0, 0, :] = pos[0].astype(jnp.int32)

        @pl.when(t == 0)
        def _blockmap():
            gstart = (lax.broadcasted_iota(jnp.int32, (1, GM), 1)
                      .astype(jnp.float32) * TT)
            cmp = (po <= gstart).astype(jnp.float32)
            n_le = jnp.sum(cmp, axis=0, keepdims=True)
            total = jnp.sum(pc)
            active = gstart < total
            eg = jnp.where(active, n_le.astype(jnp.int32) - 1, 0)
            blk_ref[pl.ds(0, 1), :] = eg
            blk_ref[pl.ds(1, 1), :] = active.astype(jnp.int32)


def _scatter_body(x_hbm, pos_hbm, out_hbm, idx_v, rows_v, sem):
    wid = lax.axis_index("s") * NC + lax.axis_index("c")
    base = wid * CHUNK
    pltpu.sync_copy(pos_hbm.at[pl.ds(base, CHUNK)], idx_v)
    pltpu.sync_copy(x_hbm.at[pl.ds(base, CHUNK)], rows_v)
    pltpu.async_copy(rows_v, out_hbm.at[idx_v], sem).wait()


def _gather_body(ys_hbm, pos_hbm, out_hbm, idx_v, rows_v, sem):
    wid = lax.axis_index("s") * NC + lax.axis_index("c")
    base = wid * CHUNK
    pltpu.sync_copy(pos_hbm.at[pl.ds(base, CHUNK)], idx_v)
    pltpu.async_copy(ys_hbm.at[idx_v], rows_v, sem).wait()
    pltpu.sync_copy(rows_v, out_hbm.at[pl.ds(base, CHUNK)])


def _sc_mesh():
    return plsc.VectorSubcoreMesh(core_axis_name="c", subcore_axis_name="s",
                                  num_cores=NC, num_subcores=NS)


def _sc_scratch():
    return [
        pltpu.VMEM((CHUNK,), jnp.int32),
        pltpu.VMEM((CHUNK, C), jnp.float32),
        pltpu.SemaphoreType.DMA,
    ]


def _sc_scatter(x_flat, pos):
    f = pl.kernel(_scatter_body,
                  out_type=jax.ShapeDtypeStruct((NPAD, C), jnp.float32),
                  mesh=_sc_mesh(), scratch_types=_sc_scratch())
    return f(x_flat, pos)


def _sc_gather(ys, pos):
    f = pl.kernel(_gather_body,
                  out_type=jax.ShapeDtypeStruct((N, C), jnp.float32),
                  mesh=_sc_mesh(), scratch_types=_sc_scratch())
    return f(ys, pos)


def _swiglu_block(xb, w1, w3, w2):
    h1 = _dot_nt(xb, w1.astype(jnp.bfloat16))
    h3 = _dot_nt(xb, w3.astype(jnp.bfloat16))
    g = (jax.nn.silu(h1) * h3).astype(jnp.bfloat16)
    return _dot_nt(g, w2.astype(jnp.bfloat16))


def _grouped_kernel(be_ref, ba_ref, xs_ref, w1_ref, w3_ref, w2_ref, ys_ref):
    g = pl.program_id(0)

    @pl.when(ba_ref[g] == 0)
    def _inactive():
        ys_ref[...] = jnp.zeros_like(ys_ref)

    @pl.when(ba_ref[g] == 1)
    def _compute():
        for ms in range(TT // MSUB):
            sl = pl.ds(ms * MSUB, MSUB)
            xb = xs_ref[sl, :].astype(jnp.bfloat16)
            ys_ref[sl, :] = _swiglu_block(xb, w1_ref[0], w3_ref[0],
                                          w2_ref[0])


def _shared_kernel(x_ref, w1_ref, w3_ref, w2_ref, o_ref):
    i = pl.program_id(0)

    @pl.when(i == 0)
    def _init():
        o_ref[...] = jnp.zeros_like(o_ref)

    xb = x_ref[...].astype(jnp.bfloat16)
    o_ref[...] += _swiglu_block(xb, w1_ref[...], w3_ref[...], w2_ref[...])


def _combine_kernel(meta_ref, sh_ref, ysu_ref, o_ref):
    ssh = meta_ref[0, 0, :]
    ssw = meta_ref[1, 0, :]
    o_ref[...] = ssh[:, None] * sh_ref[...] + ssw[:, None] * ysu_ref[...]


def kernel(x, router_w, routing_bias, sw1, sw2, sw3, ew1, ew2, ew3):
    x_flat = x.reshape(N, C)
    rb = routing_bias.reshape(E, 1)

    meta, pos3, blk = pl.pallas_call(
        _router_kernel,
        grid=(2, NT),
        in_specs=[
            pl.BlockSpec((TT, C), lambda p, t: (jnp.where(p == 0, t, NT - 1),
                                                0)),
            pl.BlockSpec((E, C), lambda p, t: (0, 0)),
            pl.BlockSpec((E, 1), lambda p, t: (0, 0)),
        ],
        out_specs=[
            pl.BlockSpec((2, 1, TT),
                         lambda p, t: (0, 0, jnp.where(p == 0, t, NT - 1))),
            pl.BlockSpec((1, 1, TT),
                         lambda p, t: (0, 0, jnp.where(p == 1, t, 0))),
            pl.BlockSpec((2, GM), lambda p, t: (0, 0)),
        ],
        out_shape=[
            jax.ShapeDtypeStruct((2, 1, N), jnp.float32),
            jax.ShapeDtypeStruct((1, 1, N), jnp.int32),
            jax.ShapeDtypeStruct((2, GM), jnp.int32),
        ],
        scratch_shapes=[
            pltpu.VMEM((E, 1), jnp.float32),
            pltpu.VMEM((1, N), jnp.int32),
            pltpu.VMEM((1, N), jnp.float32),
        ],
    )(x_flat, router_w, rb)

    pos = pos3.reshape(N)
    be = blk[0]
    ba = blk[1]

    xs = _sc_scatter(x_flat, pos)

    ys = pl.pallas_call(
        _grouped_kernel,
        grid_spec=pltpu.PrefetchScalarGridSpec(
            num_scalar_prefetch=2,
            grid=(G_MAX,),
            in_specs=[
                pl.BlockSpec((TT, C), lambda g, be, ba: (g, 0)),
                pl.BlockSpec((1, I, C), lambda g, be, ba: (be[g], 0, 0)),
                pl.BlockSpec((1, I, C), lambda g, be, ba: (be[g], 0, 0)),
                pl.BlockSpec((1, C, I), lambda g, be, ba: (be[g], 0, 0)),
            ],
            out_specs=pl.BlockSpec((TT, C), lambda g, be, ba: (g, 0)),
        ),
        out_shape=jax.ShapeDtypeStruct((NPAD, C), jnp.float32),
    )(be, ba, xs, ew1, ew3, ew2)

    shared = pl.pallas_call(
        _shared_kernel,
        grid=(NI_SH,),
        in_specs=[
            pl.BlockSpec((N, C), lambda i: (0, 0)),
            pl.BlockSpec((IB_SH, C), lambda i: (i, 0)),
            pl.BlockSpec((IB_SH, C), lambda i: (i, 0)),
            pl.BlockSpec((C, IB_SH), lambda i: (0, i)),
        ],
        out_specs=pl.BlockSpec((N, C), lambda i: (0, 0)),
        out_shape=jax.ShapeDtypeStruct((N, C), jnp.float32),
    )(x_flat, sw1, sw3, sw2)

    ysu = _sc_gather(ys, pos)

    out = pl.pallas_call(
        _combine_kernel,
        grid=(NT,),
        in_specs=[
            pl.BlockSpec((2, 1, TT), lambda t: (0, 0, t)),
            pl.BlockSpec((TT, C), lambda t: (t, 0)),
            pl.BlockSpec((TT, C), lambda t: (t, 0)),
        ],
        out_specs=pl.BlockSpec((TT, C), lambda t: (t, 0)),
        out_shape=jax.ShapeDtypeStruct((N, C), jnp.float32),
    )(meta, shared, ysu)

    return out.reshape(B, T, C)
